# baseline (speedup 1.0000x reference)
import jax
import jax.numpy as jnp
from jax.experimental import pallas as pl
from jax.experimental.pallas import tpu as pltpu


def _mm_kernel(x_ref, cb_ref, o_ref):
    o_ref[...] = jnp.dot(x_ref[...], cb_ref[...],
                         preferred_element_type=jnp.float32)


def kernel(x, codebook):
    B, N, K = x.shape
    D = codebook.shape[1]
    M = B * N
    xm = x.reshape(M, K)
    bm = 512
    out = pl.pallas_call(
        _mm_kernel,
        grid=(M // bm,),
        in_specs=[
            pl.BlockSpec((bm, K), lambda i: (i, 0)),
            pl.BlockSpec((K, D), lambda i: (0, 0)),
        ],
        out_specs=pl.BlockSpec((bm, D), lambda i: (i, 0)),
        out_shape=jax.ShapeDtypeStruct((M, D), jnp.float32),
        compiler_params=pltpu.CompilerParams(
            dimension_semantics=("parallel",),
        ),
    )(xm, codebook)
    return out.reshape(B, N, D)


# speedup vs baseline: 1.0014x; 1.0014x over previous
import jax
import jax.numpy as jnp
from jax.experimental import pallas as pl
from jax.experimental.pallas import tpu as pltpu

_SPLITS = 4
_BM = 512


def _mm_kernel(*refs):
    x_refs = refs[:_SPLITS]
    cb_ref = refs[_SPLITS]
    o_ref = refs[_SPLITS + 1]
    kq = cb_ref.shape[0] // _SPLITS
    acc = jnp.dot(x_refs[0][...], cb_ref[0:kq, :],
                  preferred_element_type=jnp.float32)
    for q in range(1, _SPLITS):
        acc = acc + jnp.dot(x_refs[q][...], cb_ref[q * kq:(q + 1) * kq, :],
                            preferred_element_type=jnp.float32)
    o_ref[...] = acc


def kernel(x, codebook):
    B, N, K = x.shape
    D = codebook.shape[1]
    M = B * N
    xm = x.reshape(M, K)
    kq = K // _SPLITS

    def x_spec(q):
        return pl.BlockSpec((_BM, kq), lambda i, q=q: (i, q))

    out = pl.pallas_call(
        _mm_kernel,
        grid=(M // _BM,),
        in_specs=[x_spec(q) for q in range(_SPLITS)] + [
            pl.BlockSpec((K, D), lambda i: (0, 0)),
        ],
        out_specs=pl.BlockSpec((_BM, D), lambda i: (i, 0)),
        out_shape=jax.ShapeDtypeStruct((M, D), jnp.float32),
        compiler_params=pltpu.CompilerParams(
            dimension_semantics=("arbitrary",),
        ),
    )(*([xm] * _SPLITS), codebook)
    return out.reshape(B, N, D)


# speedup vs baseline: 1.0034x; 1.0020x over previous
import jax
import jax.numpy as jnp
from jax.experimental import pallas as pl
from jax.experimental.pallas import tpu as pltpu


def _mm_kernel(x_ref, cb_ref, o_ref):
    xb = x_ref[...].astype(jnp.bfloat16)
    cb = cb_ref[...].astype(jnp.bfloat16)
    o_ref[...] = jnp.dot(xb, cb, preferred_element_type=jnp.float32)


def kernel(x, codebook):
    B, N, K = x.shape
    D = codebook.shape[1]
    M = B * N
    bm = 512
    out = pl.pallas_call(
        _mm_kernel,
        grid=(M // bm,),
        in_specs=[
            pl.BlockSpec((bm, K), lambda i: (i, 0)),
            pl.BlockSpec((K, D), lambda i: (0, 0)),
        ],
        out_specs=pl.BlockSpec((bm, D), lambda i: (i, 0)),
        out_shape=jax.ShapeDtypeStruct((M, D), jnp.float32),
        compiler_params=pltpu.CompilerParams(
            dimension_semantics=("arbitrary",),
        ),
    )(x.reshape(M, K), codebook)
    return out.reshape(B, N, D)
